# baseline (speedup 1.0000x reference)
import functools

import jax
import jax.numpy as jnp
from jax import lax
from jax.experimental import pallas as pl
from jax.experimental.pallas import tpu as pltpu
from jax.experimental.pallas import tpu_sc as plsc

VOCAB = 100000
NGRAM_VOCAB = 250499
EMBED = 128
HIDDEN = 256
NUM_CLASSES = 20
B = 4096
L = 50

NC = 2
NS = 16
NW = NC * NS
SPW = B // NW
NV = EMBED // 16
CHUNS = 1
ROWS = CHUNS * L
NCHUNK = SPW // CHUNS
GROUP = 4
NGROUP = NCHUNK // GROUP
UNROLL = 10


def _sc_pool():
    mesh = plsc.VectorSubcoreMesh(core_axis_name="c", subcore_axis_name="s")

    @functools.partial(
        pl.kernel,
        out_type=jax.ShapeDtypeStruct((3, B, EMBED), jnp.float32),
        mesh=mesh,
        scratch_types=[
            pltpu.VMEM((3, NCHUNK, ROWS), jnp.int32),
            pltpu.VMEM((2, GROUP * ROWS, EMBED), jnp.float32),
            pltpu.VMEM((2, GROUP * CHUNS, EMBED), jnp.float32),
            pltpu.SemaphoreType.DMA,
            pltpu.SemaphoreType.DMA,
            pltpu.SemaphoreType.DMA,
            pltpu.SemaphoreType.DMA,
        ],
    )
    def sc_pool(x, t0, t2, t3, out, idx_v, rows_v, pooled_v,
                sem0, sem1, semw0, semw1):
        wid = lax.axis_index("s") * NC + lax.axis_index("c")
        sems = (sem0, sem1)
        semws = (semw0, semw1)
        GS = GROUP * CHUNS

        def fire(t, tbl, g, buf):
            for q in range(GROUP):
                pltpu.async_copy(
                    tbl.at[idx_v.at[t, g * GROUP + q]],
                    rows_v.at[buf, pl.ds(q * ROWS, ROWS), :],
                    sems[buf])

        def drain(tbl, buf):
            pltpu.make_async_copy(
                tbl.at[pl.ds(0, GROUP * ROWS), :],
                rows_v.at[buf], sems[buf]).wait()

        def wait_write(buf):
            pltpu.make_async_copy(
                pooled_v.at[buf],
                out.at[0, pl.ds(wid * SPW, GS), :], semws[buf]).wait()

        def reduce_group(buf):
            @plsc.parallel_loop(0, GS)
            def sample_body(s):
                def red(i, acc):
                    r = s * L + UNROLL * i
                    for u in range(UNROLL):
                        acc = tuple(
                            acc[k] + rows_v[buf, r + u, pl.ds(16 * k, 16)]
                            for k in range(NV))
                    return acc

                acc = lax.fori_loop(
                    0, L // UNROLL, red,
                    tuple(jnp.zeros((16,), jnp.float32) for _ in range(NV)))
                for k in range(NV):
                    pooled_v[buf, s, pl.ds(16 * k, 16)] = acc[k]

        def write_group(t, g, buf):
            pltpu.async_copy(
                pooled_v.at[buf],
                out.at[t, pl.ds(wid * SPW + g * GS, GS), :], semws[buf])

        def handle(t, tbl, g, buf, waitw, fire_next):
            drain(tbl, buf)
            if waitw:
                wait_write(buf)
            reduce_group(buf)
            if fire_next is not None:
                nt, ntbl, ng = fire_next
                fire(nt, ntbl, ng, buf)
            write_group(t, g, buf)

        tables = (t0, t2, t3)
        for t, xi in enumerate((0, 2, 3)):
            pltpu.sync_copy(
                x.at[xi, pl.ds(wid * NCHUNK, NCHUNK), :], idx_v.at[t])

        fire(0, tables[0], 0, 0)
        fire(0, tables[0], 1, 1)
        handle(0, tables[0], 0, 0, False, (0, tables[0], 2))
        handle(0, tables[0], 1, 1, False, (0, tables[0], 3))

        for t in range(3):
            tbl = tables[t]

            def make_pair_body(t, tbl):
                def pair_body(i, _):
                    for buf in range(2):
                        handle(t, tbl, 2 * i + buf, buf, True,
                               (t, tbl, 2 * i + buf + 2))
                    return 0
                return pair_body

            lo = 1 if t == 0 else 0
            lax.fori_loop(lo, (NGROUP - 2) // 2, make_pair_body(t, tbl), 0)

            if t < 2:
                nxt = ((t + 1, tables[t + 1], 0), (t + 1, tables[t + 1], 1))
            else:
                nxt = (None, None)
            handle(t, tbl, NGROUP - 2, 0, True, nxt[0])
            handle(t, tbl, NGROUP - 1, 1, True, nxt[1])

        wait_write(0)
        wait_write(1)

    return sc_pool


_SC_POOL = _sc_pool()


def _mlp_body(p_ref, w1_ref, b1_ref, w2_ref, b2_ref, out_ref):
    p = p_ref[...]
    h = jnp.dot(p[0], w1_ref[0:EMBED, :], preferred_element_type=jnp.float32)
    h += jnp.dot(p[1], w1_ref[EMBED:2 * EMBED, :],
                 preferred_element_type=jnp.float32)
    h += jnp.dot(p[2], w1_ref[2 * EMBED:3 * EMBED, :],
                 preferred_element_type=jnp.float32)
    h = h * (1.0 / L) + b1_ref[...]
    h = jnp.maximum(h, 0.0)
    y = jnp.dot(h, w2_ref[...], preferred_element_type=jnp.float32)
    out_ref[...] = y + b2_ref[...]


def _mlp(pooled, W1, b1, W2, b2):
    bs = 512
    grid = (B // bs,)
    return pl.pallas_call(
        _mlp_body,
        grid=grid,
        in_specs=[
            pl.BlockSpec((3, bs, EMBED), lambda i: (0, i, 0)),
            pl.BlockSpec((3 * EMBED, HIDDEN), lambda i: (0, 0)),
            pl.BlockSpec((1, HIDDEN), lambda i: (0, 0)),
            pl.BlockSpec((HIDDEN, NUM_CLASSES), lambda i: (0, 0)),
            pl.BlockSpec((1, NUM_CLASSES), lambda i: (0, 0)),
        ],
        out_specs=pl.BlockSpec((bs, NUM_CLASSES), lambda i: (i, 0)),
        out_shape=jax.ShapeDtypeStruct((B, NUM_CLASSES), jnp.float32),
    )(pooled, W1, b1, W2, b2)


def kernel(x, emb_word, emb2, emb3, W1, b1, W2, b2):
    x = x.astype(jnp.int32).reshape(4, B * L // ROWS, ROWS)
    pooled = _SC_POOL(x, emb_word, emb2, emb3)
    return _mlp(pooled, W1, b1.reshape(1, HIDDEN), W2,
                b2.reshape(1, NUM_CLASSES))


# speedup vs baseline: 1.0659x; 1.0659x over previous
import functools

import jax
import jax.numpy as jnp
from jax import lax
from jax.experimental import pallas as pl
from jax.experimental.pallas import tpu as pltpu
from jax.experimental.pallas import tpu_sc as plsc

VOCAB = 100000
NGRAM_VOCAB = 250499
EMBED = 128
HIDDEN = 256
NUM_CLASSES = 20
B = 4096
L = 50

NC = 2
NS = 16
NW = NC * NS
SPW = B // NW
NV = EMBED // 16
CHUNS = 2
ROWS = CHUNS * L
NCHUNK = SPW // CHUNS
GROUP = 4
NGROUP = NCHUNK // GROUP
UNROLL = 10


def _sc_pool():
    mesh = plsc.VectorSubcoreMesh(core_axis_name="c", subcore_axis_name="s")

    @functools.partial(
        pl.kernel,
        out_type=jax.ShapeDtypeStruct((3, B, EMBED), jnp.float32),
        mesh=mesh,
        scratch_types=[
            pltpu.VMEM((3, NCHUNK, ROWS), jnp.int32),
            pltpu.VMEM((2, GROUP * ROWS, EMBED), jnp.float32),
            pltpu.VMEM((2, GROUP * CHUNS, EMBED), jnp.float32),
            pltpu.SemaphoreType.DMA,
            pltpu.SemaphoreType.DMA,
            pltpu.SemaphoreType.DMA,
            pltpu.SemaphoreType.DMA,
        ],
    )
    def sc_pool(x, t0, t2, t3, out, idx_v, rows_v, pooled_v,
                sem0, sem1, semw0, semw1):
        wid = lax.axis_index("s") * NC + lax.axis_index("c")
        sems = (sem0, sem1)
        semws = (semw0, semw1)
        GS = GROUP * CHUNS

        def fire(t, tbl, g, buf):
            for q in range(GROUP):
                pltpu.async_copy(
                    tbl.at[idx_v.at[t, g * GROUP + q]],
                    rows_v.at[buf, pl.ds(q * ROWS, ROWS), :],
                    sems[buf])

        def drain(tbl, buf):
            pltpu.make_async_copy(
                tbl.at[pl.ds(0, GROUP * ROWS), :],
                rows_v.at[buf], sems[buf]).wait()

        def wait_write(buf):
            pltpu.make_async_copy(
                pooled_v.at[buf],
                out.at[0, pl.ds(wid * SPW, GS), :], semws[buf]).wait()

        def reduce_group(buf):
            @plsc.parallel_loop(0, GS)
            def sample_body(s):
                def red(i, acc):
                    r = s * L + UNROLL * i
                    for u in range(UNROLL):
                        acc = tuple(
                            acc[k] + rows_v[buf, r + u, pl.ds(16 * k, 16)]
                            for k in range(NV))
                    return acc

                acc = lax.fori_loop(
                    0, L // UNROLL, red,
                    tuple(jnp.zeros((16,), jnp.float32) for _ in range(NV)))
                for k in range(NV):
                    pooled_v[buf, s, pl.ds(16 * k, 16)] = acc[k]

        def write_group(t, g, buf):
            pltpu.async_copy(
                pooled_v.at[buf],
                out.at[t, pl.ds(wid * SPW + g * GS, GS), :], semws[buf])

        def handle(t, tbl, g, buf, waitw, fire_next):
            drain(tbl, buf)
            if waitw:
                wait_write(buf)
            reduce_group(buf)
            if fire_next is not None:
                nt, ntbl, ng = fire_next
                fire(nt, ntbl, ng, buf)
            write_group(t, g, buf)

        tables = (t0, t2, t3)
        for t, xi in enumerate((0, 2, 3)):
            pltpu.sync_copy(
                x.at[xi, pl.ds(wid * NCHUNK, NCHUNK), :], idx_v.at[t])

        fire(0, tables[0], 0, 0)
        fire(0, tables[0], 1, 1)
        handle(0, tables[0], 0, 0, False, (0, tables[0], 2))
        handle(0, tables[0], 1, 1, False, (0, tables[0], 3))

        for t in range(3):
            tbl = tables[t]

            def make_pair_body(t, tbl):
                def pair_body(i, _):
                    for buf in range(2):
                        handle(t, tbl, 2 * i + buf, buf, True,
                               (t, tbl, 2 * i + buf + 2))
                    return 0
                return pair_body

            lo = 1 if t == 0 else 0
            lax.fori_loop(lo, (NGROUP - 2) // 2, make_pair_body(t, tbl), 0)

            if t < 2:
                nxt = ((t + 1, tables[t + 1], 0), (t + 1, tables[t + 1], 1))
            else:
                nxt = (None, None)
            handle(t, tbl, NGROUP - 2, 0, True, nxt[0])
            handle(t, tbl, NGROUP - 1, 1, True, nxt[1])

        wait_write(0)
        wait_write(1)

    return sc_pool


_SC_POOL = _sc_pool()


def _mlp_body(p_ref, w1_ref, b1_ref, w2_ref, b2_ref, out_ref):
    p = p_ref[...]
    h = jnp.dot(p[0], w1_ref[0:EMBED, :], preferred_element_type=jnp.float32)
    h += jnp.dot(p[1], w1_ref[EMBED:2 * EMBED, :],
                 preferred_element_type=jnp.float32)
    h += jnp.dot(p[2], w1_ref[2 * EMBED:3 * EMBED, :],
                 preferred_element_type=jnp.float32)
    h = h * (1.0 / L) + b1_ref[...]
    h = jnp.maximum(h, 0.0)
    y = jnp.dot(h, w2_ref[...], preferred_element_type=jnp.float32)
    out_ref[...] = y + b2_ref[...]


def _mlp(pooled, W1, b1, W2, b2):
    bs = 512
    grid = (B // bs,)
    return pl.pallas_call(
        _mlp_body,
        grid=grid,
        in_specs=[
            pl.BlockSpec((3, bs, EMBED), lambda i: (0, i, 0)),
            pl.BlockSpec((3 * EMBED, HIDDEN), lambda i: (0, 0)),
            pl.BlockSpec((1, HIDDEN), lambda i: (0, 0)),
            pl.BlockSpec((HIDDEN, NUM_CLASSES), lambda i: (0, 0)),
            pl.BlockSpec((1, NUM_CLASSES), lambda i: (0, 0)),
        ],
        out_specs=pl.BlockSpec((bs, NUM_CLASSES), lambda i: (i, 0)),
        out_shape=jax.ShapeDtypeStruct((B, NUM_CLASSES), jnp.float32),
    )(pooled, W1, b1, W2, b2)


def kernel(x, emb_word, emb2, emb3, W1, b1, W2, b2):
    x = x.astype(jnp.int32).reshape(4, B * L // ROWS, ROWS)
    pooled = _SC_POOL(x, emb_word, emb2, emb3)
    return _mlp(pooled, W1, b1.reshape(1, HIDDEN), W2,
                b2.reshape(1, NUM_CLASSES))
